# baseline (speedup 1.0000x reference)
import functools

import jax
import jax.numpy as jnp
from jax.experimental import pallas as pl

_BLK = 1000


def _mlp_block_kernel(x_ref, wn_ref, bn_ref, wo_ref, bo_ref, o_ref):
    h = jnp.dot(x_ref[...], wn_ref[...], preferred_element_type=jnp.float32)
    h = jnp.maximum(h + bn_ref[...], 0.0)
    o = jnp.dot(h, wo_ref[...], preferred_element_type=jnp.float32)
    o_ref[...] = o + bo_ref[...]


@functools.partial(jax.jit, static_argnames=())
def _fused_mlp(x, W_node, b_node, W_out, b_out):
    n, d = x.shape
    d_out = W_out.shape[1]
    grid = (n // _BLK,)
    return pl.pallas_call(
        _mlp_block_kernel,
        grid=grid,
        in_specs=[
            pl.BlockSpec((_BLK, d), lambda i: (i, 0)),
            pl.BlockSpec((d, W_node.shape[1]), lambda i: (0, 0)),
            pl.BlockSpec((1, W_node.shape[1]), lambda i: (0, 0)),
            pl.BlockSpec((W_node.shape[1], d_out), lambda i: (0, 0)),
            pl.BlockSpec((1, d_out), lambda i: (0, 0)),
        ],
        out_specs=pl.BlockSpec((_BLK, d_out), lambda i: (i, 0)),
        out_shape=jax.ShapeDtypeStruct((n, d_out), jnp.float32),
    )(x, W_node, b_node.reshape(1, -1), W_out, b_out.reshape(1, -1))


def kernel(x, edge_index, edge_weight, W_node, b_node, W_edge, b_edge, W_out, b_out):
    return _fused_mlp(x, W_node, b_node, W_out, b_out)
